# Initial kernel scaffold; baseline (speedup 1.0000x reference)
import functools

import jax
import jax.numpy as jnp
from jax import lax
from jax.experimental import pallas as pl
from jax.experimental.pallas import tpu as pltpu
from jax.experimental.pallas import tpu_sc as plsc

NC = 2
NS = 16
NW = NC * NS
L = 16

D = 128
B = 4096 * 50
B_PER_W = B // NW
CHUNK = 128
NCHUNK = B_PER_W // CHUNK
IDX_ROWS = NCHUNK


def _worker_body(table, xr, out, idx_v, rows_v, gsem):
    wid = lax.axis_index("s") * NC + lax.axis_index("c")
    idx_row0 = wid * IDX_ROWS
    out_base = wid * B_PER_W

    pltpu.sync_copy(xr.at[pl.ds(idx_row0, IDX_ROWS)], idx_v)

    @pl.loop(0, NCHUNK)
    def _chunk(c):
        pltpu.async_copy(table.at[idx_v.at[c]], rows_v, gsem).wait()

        vs = [idx_v[c, pl.ds(j * L, L)] for j in range(CHUNK // L)]
        mn = vs[0]
        for v in vs[1:]:
            mn = jnp.minimum(mn, v)
        chunk_min = jnp.min(mn)

        @pl.when(chunk_min == 0)
        def _fix():
            zeros = jnp.zeros((L,), jnp.float32)
            base_iota = lax.iota(jnp.int32, L)
            for j in range(CHUNK // L):
                vj = idx_v[c, pl.ds(j * L, L)]
                msk = vj == 0

                @pl.when(jnp.min(vj) == 0)
                def _group():
                    rows = base_iota + (j * L)

                    @pl.loop(0, D)
                    def _col(col):
                        colv = jnp.full((L,), col, jnp.int32)
                        plsc.store_scatter(rows_v, [rows, colv], zeros,
                                           mask=msk)

        pltpu.sync_copy(rows_v, out.at[pl.ds(out_base + c * CHUNK, CHUNK)])


@jax.jit
def kernel(embedding, x):
    xr = x.astype(jnp.int32).reshape(NW * IDX_ROWS, CHUNK)
    mesh = plsc.VectorSubcoreMesh(
        core_axis_name="c", subcore_axis_name="s",
        num_cores=NC, num_subcores=NS,
    )
    out = pl.kernel(
        _worker_body,
        out_type=jax.ShapeDtypeStruct((B, D), jnp.float32),
        mesh=mesh,
        scratch_types=[
            pltpu.VMEM((IDX_ROWS, CHUNK), jnp.int32),
            pltpu.VMEM((CHUNK, D), jnp.float32),
            pltpu.SemaphoreType.DMA,
        ],
    )(embedding, xr)
    return out.reshape(x.shape + (D,))


# speedup vs baseline: 2.9472x; 2.9472x over previous
import functools

import jax
import jax.numpy as jnp
from jax import lax
from jax.experimental import pallas as pl
from jax.experimental.pallas import tpu as pltpu
from jax.experimental.pallas import tpu_sc as plsc

NC = 2
NS = 16
NW = NC * NS
L = 16

D = 128
B = 4096 * 50
B_PER_W = B // NW
CHUNK = 128
NCHUNK = B_PER_W // CHUNK
IDX_ROWS = NCHUNK


def _worker_body(table, xr, out, idx_v, rows_v, gsem):
    wid = lax.axis_index("s") * NC + lax.axis_index("c")
    out_base = wid * B_PER_W

    pltpu.sync_copy(xr.at[wid], idx_v)

    @pl.loop(0, NCHUNK)
    def _chunk(c):
        pltpu.async_copy(table.at[idx_v.at[c]], rows_v, gsem).wait()

        vs = [idx_v[c, pl.ds(j * L, L)] for j in range(CHUNK // L)]
        mn = vs[0]
        for v in vs[1:]:
            mn = jnp.minimum(mn, v)
        cnt = plsc.all_reduce_population_count(mn == 0)
        has_pad = cnt[0] > 0

        @pl.when(has_pad)
        def _fix():
            zeros = jnp.zeros((L,), jnp.float32)
            base_iota = lax.iota(jnp.int32, L)
            for j in range(CHUNK // L):
                vj = idx_v[c, pl.ds(j * L, L)]
                msk = vj == 0
                rows = base_iota + (j * L)

                @pl.loop(0, D)
                def _col(col):
                    colv = jnp.full((L,), col, jnp.int32)
                    plsc.store_scatter(rows_v, [rows, colv], zeros,
                                       mask=msk)

        pltpu.sync_copy(rows_v, out.at[pl.ds(out_base + c * CHUNK, CHUNK)])


@jax.jit
def kernel(embedding, x):
    xr = x.astype(jnp.int32).reshape(NW, IDX_ROWS, CHUNK)
    mesh = plsc.VectorSubcoreMesh(
        core_axis_name="c", subcore_axis_name="s",
        num_cores=NC, num_subcores=NS,
    )
    out = pl.kernel(
        _worker_body,
        out_type=jax.ShapeDtypeStruct((B, D), jnp.float32),
        mesh=mesh,
        compiler_params=pltpu.CompilerParams(needs_layout_passes=False),
        scratch_types=[
            pltpu.VMEM((IDX_ROWS, CHUNK), jnp.int32),
            pltpu.VMEM((CHUNK, D), jnp.float32),
            pltpu.SemaphoreType.DMA,
        ],
    )(embedding, xr)
    return out.reshape(x.shape + (D,))


# speedup vs baseline: 3.1049x; 1.0535x over previous
import functools

import jax
import jax.numpy as jnp
from jax import lax
from jax.experimental import pallas as pl
from jax.experimental.pallas import tpu as pltpu
from jax.experimental.pallas import tpu_sc as plsc

NC = 2
NS = 16
NW = NC * NS
L = 16

D = 128
B = 4096 * 50
B_PER_W = B // NW
CHUNK = 128
NCHUNK = B_PER_W // CHUNK
IDX_ROWS = NCHUNK


def _worker_body(table, xr, out, idx_v, buf0, buf1, g0, g1, s0, s1):
    wid = lax.axis_index("s") * NC + lax.axis_index("c")
    out_base = wid * B_PER_W
    bufs = (buf0, buf1)
    gsems = (g0, g1)
    ssems = (s0, s1)

    pltpu.sync_copy(xr.at[wid], idx_v)

    def start_gather(c, k):
        pltpu.async_copy(table.at[idx_v.at[c]], bufs[k], gsems[k])

    def wait_gather(c, k):
        pltpu.make_async_copy(table.at[idx_v.at[c]], bufs[k],
                              gsems[k]).wait()

    def out_slice(c):
        return out.at[pl.ds(out_base + c * CHUNK, CHUNK)]

    def start_scatter(c, k):
        pltpu.async_copy(bufs[k], out_slice(c), ssems[k])

    def wait_scatter(c, k):
        pltpu.make_async_copy(bufs[k], out_slice(c), ssems[k]).wait()

    def fix(c, k):
        rows_v = bufs[k]
        vs = [idx_v[c, pl.ds(j * L, L)] for j in range(CHUNK // L)]
        mn = vs[0]
        for v in vs[1:]:
            mn = jnp.minimum(mn, v)
        cnt = plsc.all_reduce_population_count(mn == 0)
        has_pad = cnt[0] > 0

        @pl.when(has_pad)
        def _fix():
            zeros = jnp.zeros((L,), jnp.float32)
            base_iota = lax.iota(jnp.int32, L)
            for j in range(CHUNK // L):
                vj = idx_v[c, pl.ds(j * L, L)]
                msk = vj == 0
                rows = base_iota + (j * L)

                @pl.loop(0, D)
                def _col(col):
                    colv = jnp.full((L,), col, jnp.int32)
                    plsc.store_scatter(rows_v, [rows, colv], zeros,
                                       mask=msk)

    def process(c, k):
        wait_gather(c, k)
        fix(c, k)
        start_scatter(c, k)

    start_gather(0, 0)
    process(0, 0)
    start_gather(1, 1)

    @pl.loop(0, (NCHUNK - 2) // 2)
    def _steady(i):
        c1 = 2 * i + 1
        process(c1, 1)
        wait_scatter(c1 - 1, 0)
        start_gather(c1 + 1, 0)
        c2 = 2 * i + 2
        process(c2, 0)
        wait_scatter(c2 - 1, 1)
        start_gather(c2 + 1, 1)

    process(NCHUNK - 1, 1)
    wait_scatter(NCHUNK - 2, 0)
    wait_scatter(NCHUNK - 1, 1)


@jax.jit
def kernel(embedding, x):
    xr = x.astype(jnp.int32).reshape(NW, IDX_ROWS, CHUNK)
    mesh = plsc.VectorSubcoreMesh(
        core_axis_name="c", subcore_axis_name="s",
        num_cores=NC, num_subcores=NS,
    )
    out = pl.kernel(
        _worker_body,
        out_type=jax.ShapeDtypeStruct((B, D), jnp.float32),
        mesh=mesh,
        compiler_params=pltpu.CompilerParams(needs_layout_passes=False),
        scratch_types=[
            pltpu.VMEM((IDX_ROWS, CHUNK), jnp.int32),
            pltpu.VMEM((CHUNK, D), jnp.float32),
            pltpu.VMEM((CHUNK, D), jnp.float32),
            pltpu.SemaphoreType.DMA,
            pltpu.SemaphoreType.DMA,
            pltpu.SemaphoreType.DMA,
            pltpu.SemaphoreType.DMA,
        ],
    )(embedding, xr)
    return out.reshape(x.shape + (D,))


# speedup vs baseline: 5.7500x; 1.8519x over previous
import jax
import jax.numpy as jnp
from jax import lax
from jax.experimental import pallas as pl
from jax.experimental.pallas import tpu as pltpu
from jax.experimental.pallas import tpu_sc as plsc

NC = 2
NS = 16
NW = NC * NS
L = 16

D = 128
S = 4096
T = 50
S_PER_W = S // NW
CH_S = 8
CH_ROWS = CH_S * T
NCHUNK = S_PER_W // CH_S

_GROUP_OFF = (0, 16, 32, T - L)


def _worker_body(table, x, out, idx_v, buf0, buf1, g0, g1, s0, s1):
    wid = lax.axis_index("s") * NC + lax.axis_index("c")
    sent0 = wid * S_PER_W
    bufs = (buf0, buf1)
    gsems = (g0, g1)
    ssems = (s0, s1)

    pltpu.sync_copy(x.at[pl.ds(sent0, S_PER_W)], idx_v)

    def gather_parts(c, k):
        for j in range(CH_S):
            yield (table.at[idx_v.at[c * CH_S + j]],
                   bufs[k].at[pl.ds(j * T, T)], gsems[k])

    def start_gather(c, k):
        for src, dst, sem in gather_parts(c, k):
            pltpu.async_copy(src, dst, sem)

    def wait_gather(c, k):
        for src, dst, sem in gather_parts(c, k):
            pltpu.make_async_copy(src, dst, sem).wait()

    def scatter_parts(c, k):
        for j in range(CH_S):
            yield (bufs[k].at[pl.ds(j * T, T)],
                   out.at[sent0 + c * CH_S + j], ssems[k])

    def start_scatter(c, k):
        for src, dst, sem in scatter_parts(c, k):
            pltpu.async_copy(src, dst, sem)

    def wait_scatter(c, k):
        for src, dst, sem in scatter_parts(c, k):
            pltpu.make_async_copy(src, dst, sem).wait()

    def sent_idx_vecs(sl):
        return [idx_v[sl, pl.ds(off, L)] for off in _GROUP_OFF]

    def fix(c, k):
        rows_v = bufs[k]
        mns = []
        for j in range(CH_S):
            mns.extend(sent_idx_vecs(c * CH_S + j))
        mn = mns[0]
        for v in mns[1:]:
            mn = jnp.minimum(mn, v)
        has_pad = plsc.all_reduce_population_count(mn == 0)[0] > 0

        @pl.when(has_pad)
        def _fix_chunk():
            zeros = jnp.zeros((L,), jnp.float32)
            base_iota = lax.iota(jnp.int32, L)
            for j in range(CH_S):
                vs = sent_idx_vecs(c * CH_S + j)
                smn = vs[0]
                for v in vs[1:]:
                    smn = jnp.minimum(smn, v)
                sent_pad = plsc.all_reduce_population_count(smn == 0)[0] > 0

                @pl.when(sent_pad)
                def _fix_sent():
                    for g, off in enumerate(_GROUP_OFF):
                        msk = vs[g] == 0
                        rows = base_iota + (j * T + off)

                        @pl.loop(0, D)
                        def _col(col):
                            colv = jnp.full((L,), col, jnp.int32)
                            plsc.store_scatter(rows_v, [rows, colv],
                                               zeros, mask=msk)

    def process(c, k):
        wait_gather(c, k)
        fix(c, k)
        start_scatter(c, k)

    start_gather(0, 0)
    process(0, 0)
    start_gather(1, 1)

    @pl.loop(0, (NCHUNK - 2) // 2)
    def _steady(i):
        c1 = 2 * i + 1
        process(c1, 1)
        wait_scatter(c1 - 1, 0)
        start_gather(c1 + 1, 0)
        c2 = 2 * i + 2
        process(c2, 0)
        wait_scatter(c2 - 1, 1)
        start_gather(c2 + 1, 1)

    process(NCHUNK - 1, 1)
    wait_scatter(NCHUNK - 2, 0)
    wait_scatter(NCHUNK - 1, 1)


@jax.jit
def kernel(embedding, x):
    xi = x.astype(jnp.int32)
    mesh = plsc.VectorSubcoreMesh(
        core_axis_name="c", subcore_axis_name="s",
        num_cores=NC, num_subcores=NS,
    )
    return pl.kernel(
        _worker_body,
        out_type=jax.ShapeDtypeStruct((S, T, D), jnp.float32),
        mesh=mesh,
        compiler_params=pltpu.CompilerParams(needs_layout_passes=False),
        scratch_types=[
            pltpu.VMEM((S_PER_W, T), jnp.int32),
            pltpu.VMEM((CH_ROWS, D), jnp.float32),
            pltpu.VMEM((CH_ROWS, D), jnp.float32),
            pltpu.SemaphoreType.DMA,
            pltpu.SemaphoreType.DMA,
            pltpu.SemaphoreType.DMA,
            pltpu.SemaphoreType.DMA,
        ],
    )(embedding, xi)
